# Initial kernel scaffold; baseline (speedup 1.0000x reference)
import jax
import jax.numpy as jnp
from jax.experimental import pallas as pl


def kernel(x, W, b, bias):
    raise NotImplementedError("write your pallas kernel here")


# speedup vs baseline: 1.1385x; 1.1385x over previous
import jax
import jax.numpy as jnp
from jax import lax
from jax.experimental import pallas as pl

TOPK = 8
NG = 8
GSZ = 8
KG = 4
NE = 64
DIN = 1024


def _gate_block(x_ref, w_ref, b_ref, bias_ref, wout_ref, iout_ref):
    xb = x_ref[...]
    W = w_ref[...]
    s_lin = lax.dot_general(xb, W, (((1,), (1,)), ((), ())),
                            preferred_element_type=jnp.float32)
    s_lin = s_lin + b_ref[...]
    s2w = jax.nn.sigmoid(s_lin)
    score = s2w + bias_ref[...]
    bB = score.shape[0]
    lane = lax.broadcasted_iota(jnp.int32, (bB, NE), 1)
    group = lane // GSZ
    neg = jnp.float32(-jnp.inf)

    gcols = []
    for g in range(NG):
        in_g = group == g
        sg = jnp.where(in_g, score, neg)
        m1 = jnp.max(sg, axis=1, keepdims=True)
        l1 = jnp.min(jnp.where(sg == m1, lane, NE), axis=1, keepdims=True)
        m2 = jnp.max(jnp.where(lane == l1, neg, sg), axis=1, keepdims=True)
        gcols.append(m1 + m2)
    gs = jnp.concatenate(gcols, axis=1)

    g_iota = lax.broadcasted_iota(jnp.int32, (bB, NG), 1)
    rank = jnp.zeros((bB, NG), jnp.int32)
    for h in range(NG):
        gh = gs[:, h:h + 1]
        rank = rank + (gh > gs).astype(jnp.int32)
        rank = rank + ((gh == gs) & (h < g_iota)).astype(jnp.int32)
    keep = (rank < KG).astype(jnp.float32)
    mask = jnp.zeros((bB, NE), jnp.float32)
    for g in range(NG):
        mask = mask + keep[:, g:g + 1] * (group == g).astype(jnp.float32)
    score_f = score * mask
    s2w_f = s2w * mask

    cur = score_f
    wcols, icols = [], []
    for _ in range(TOPK):
        m = jnp.max(cur, axis=1, keepdims=True)
        lsel = jnp.min(jnp.where(cur == m, lane, NE), axis=1, keepdims=True)
        hit = lane == lsel
        wcols.append(jnp.max(jnp.where(hit, s2w_f, neg), axis=1, keepdims=True))
        icols.append(lsel)
        cur = jnp.where(hit, neg, cur)
    wout_ref[...] = jnp.concatenate(wcols, axis=1)
    iout_ref[...] = jnp.concatenate(icols, axis=1)


def kernel(x, W, b, bias):
    B = x.shape[0]
    bB = 256
    b2 = b.reshape(1, NE)
    wout, iout = pl.pallas_call(
        _gate_block,
        grid=(B // bB,),
        in_specs=[
            pl.BlockSpec((bB, DIN), lambda i: (i, 0)),
            pl.BlockSpec((NE, DIN), lambda i: (0, 0)),
            pl.BlockSpec((1, NE), lambda i: (0, 0)),
            pl.BlockSpec((1, NE), lambda i: (0, 0)),
        ],
        out_specs=[
            pl.BlockSpec((bB, TOPK), lambda i: (i, 0)),
            pl.BlockSpec((bB, TOPK), lambda i: (i, 0)),
        ],
        out_shape=[
            jax.ShapeDtypeStruct((B, TOPK), jnp.float32),
            jax.ShapeDtypeStruct((B, TOPK), jnp.int32),
        ],
    )(x, W, b2, bias)
    return wout, iout


# speedup vs baseline: 6.4994x; 5.7085x over previous
import jax
import jax.numpy as jnp
from jax import lax
from jax.experimental import pallas as pl

TOPK = 8
NG = 8
GSZ = 8
KG = 4
NE = 64
DIN = 1024


def _top2_merge(m1a, m2a, m1b, m2b):
    return (jnp.maximum(m1a, m1b),
            jnp.maximum(jnp.minimum(m1a, m1b), jnp.maximum(m2a, m2b)))


def _gate_block(x_ref, w_ref, b_ref, bias_ref, wout_ref, iout_ref):
    xb = x_ref[...]
    W = w_ref[...]
    s_lin = lax.dot_general(W, xb, (((1,), (1,)), ((), ())),
                            preferred_element_type=jnp.float32)
    s_lin = s_lin + b_ref[...]
    s2w = jax.nn.sigmoid(s_lin)
    score = s2w + bias_ref[...]
    bB = score.shape[1]
    neg = jnp.float32(-jnp.inf)

    gs_rows = []
    for g in range(NG):
        v = score[g * GSZ:(g + 1) * GSZ]
        m1, m2 = _top2_merge(v[0:4], jnp.full_like(v[0:4], neg),
                             v[4:8], jnp.full_like(v[0:4], neg))
        m1, m2 = _top2_merge(m1[0:2], m2[0:2], m1[2:4], m2[2:4])
        m1, m2 = _top2_merge(m1[0:1], m2[0:1], m1[1:2], m2[1:2])
        gs_rows.append(m1 + m2)
    gs = jnp.concatenate(gs_rows, axis=0)

    rowg = lax.broadcasted_iota(jnp.int32, (NG, bB), 0)
    rank = jnp.zeros((NG, bB), jnp.int32)
    for h in range(NG):
        gh = gs[h:h + 1]
        rank = rank + (gh > gs).astype(jnp.int32)
        rank = rank + ((gh == gs) & (h < rowg)).astype(jnp.int32)
    keep = (rank < KG).astype(jnp.float32)
    mask = jnp.concatenate(
        [jnp.broadcast_to(keep[g:g + 1], (GSZ, bB)) for g in range(NG)],
        axis=0)
    score_f = score * mask
    s2w_f = s2w * mask

    row = lax.broadcasted_iota(jnp.int32, (NE, bB), 0)
    cur = score_f
    wrows, irows = [], []
    for _ in range(TOPK):
        m = jnp.max(cur, axis=0, keepdims=True)
        lsel = jnp.min(jnp.where(cur == m, row, NE), axis=0, keepdims=True)
        hit = row == lsel
        wrows.append(jnp.max(jnp.where(hit, s2w_f, neg), axis=0, keepdims=True))
        irows.append(lsel)
        cur = jnp.where(hit, neg, cur)
    wout_ref[...] = jnp.concatenate(wrows, axis=0)
    iout_ref[...] = jnp.concatenate(irows, axis=0)


def kernel(x, W, b, bias):
    B = x.shape[0]
    bB = 256
    b2 = b.reshape(NE, 1)
    bias2 = bias.reshape(NE, 1)
    wout, iout = pl.pallas_call(
        _gate_block,
        grid=(B // bB,),
        in_specs=[
            pl.BlockSpec((bB, DIN), lambda i: (i, 0)),
            pl.BlockSpec((NE, DIN), lambda i: (0, 0)),
            pl.BlockSpec((NE, 1), lambda i: (0, 0)),
            pl.BlockSpec((NE, 1), lambda i: (0, 0)),
        ],
        out_specs=[
            pl.BlockSpec((TOPK, bB), lambda i: (0, i)),
            pl.BlockSpec((TOPK, bB), lambda i: (0, i)),
        ],
        out_shape=[
            jax.ShapeDtypeStruct((TOPK, B), jnp.float32),
            jax.ShapeDtypeStruct((TOPK, B), jnp.int32),
        ],
    )(x, W, b2, bias2)
    return wout.T, iout.T


# speedup vs baseline: 9.1584x; 1.4091x over previous
import jax
import jax.numpy as jnp
from jax import lax
from jax.experimental import pallas as pl

TOPK = 8
NG = 8
GSZ = 8
KG = 4
NE = 64
DIN = 1024


def _top2_merge(m1a, m2a, m1b, m2b):
    return (jnp.maximum(m1a, m1b),
            jnp.maximum(jnp.minimum(m1a, m1b), jnp.maximum(m2a, m2b)))


def _gate_block(x_ref, w_ref, b_ref, bias_ref, wout_ref, iout_ref):
    xb = x_ref[...]
    W = w_ref[...]
    s_lin = lax.dot_general(W, xb, (((1,), (1,)), ((), ())),
                            preferred_element_type=jnp.float32)
    s_lin = s_lin + b_ref[...]
    s2w = jax.nn.sigmoid(s_lin)
    score = s2w + bias_ref[...]
    bB = score.shape[1]
    neg = jnp.float32(-jnp.inf)

    gs_rows = []
    for g in range(NG):
        v = score[g * GSZ:(g + 1) * GSZ]
        m1, m2 = _top2_merge(v[0:4], jnp.full_like(v[0:4], neg),
                             v[4:8], jnp.full_like(v[0:4], neg))
        m1, m2 = _top2_merge(m1[0:2], m2[0:2], m1[2:4], m2[2:4])
        m1, m2 = _top2_merge(m1[0:1], m2[0:1], m1[1:2], m2[1:2])
        gs_rows.append(m1 + m2)
    gs = jnp.concatenate(gs_rows, axis=0)

    rowg = lax.broadcasted_iota(jnp.int32, (NG, bB), 0)
    rank = jnp.zeros((NG, bB), jnp.int32)
    for h in range(NG):
        gh = gs[h:h + 1]
        rank = rank + (gh > gs).astype(jnp.int32)
        rank = rank + ((gh == gs) & (h < rowg)).astype(jnp.int32)
    keep = (rank < KG).astype(jnp.float32)
    mask = jnp.concatenate(
        [jnp.broadcast_to(keep[g:g + 1], (GSZ, bB)) for g in range(NG)],
        axis=0)
    score_f = score * mask
    s2w_f = s2w * mask

    row = lax.broadcasted_iota(jnp.int32, (NE, bB), 0)
    cur = score_f
    wrows, irows = [], []
    for _ in range(TOPK):
        m = jnp.max(cur, axis=0, keepdims=True)
        lsel = jnp.min(jnp.where(cur == m, row, NE), axis=0, keepdims=True)
        hit = row == lsel
        wrows.append(jnp.max(jnp.where(hit, s2w_f, neg), axis=0, keepdims=True))
        irows.append(lsel)
        cur = jnp.where(hit, neg, cur)
    wout_ref[...] = jnp.concatenate(wrows, axis=0)
    iout_ref[...] = jnp.concatenate(irows, axis=0)


def kernel(x, W, b, bias):
    B = x.shape[0]
    bB = 512
    b2 = b.reshape(NE, 1)
    bias2 = bias.reshape(NE, 1)
    wout, iout = pl.pallas_call(
        _gate_block,
        grid=(B // bB,),
        in_specs=[
            pl.BlockSpec((bB, DIN), lambda i: (i, 0)),
            pl.BlockSpec((NE, DIN), lambda i: (0, 0)),
            pl.BlockSpec((NE, 1), lambda i: (0, 0)),
            pl.BlockSpec((NE, 1), lambda i: (0, 0)),
        ],
        out_specs=[
            pl.BlockSpec((TOPK, bB), lambda i: (0, i)),
            pl.BlockSpec((TOPK, bB), lambda i: (0, i)),
        ],
        out_shape=[
            jax.ShapeDtypeStruct((TOPK, B), jnp.float32),
            jax.ShapeDtypeStruct((TOPK, B), jnp.int32),
        ],
    )(x, W, b2, bias2)
    return wout.T, iout.T


# speedup vs baseline: 11.8238x; 1.2910x over previous
import jax
import jax.numpy as jnp
from jax import lax
from jax.experimental import pallas as pl

TOPK = 8
NG = 8
GSZ = 8
KG = 4
NE = 64
DIN = 1024


def _top2_merge(m1a, m2a, m1b, m2b):
    return (jnp.maximum(m1a, m1b),
            jnp.maximum(jnp.minimum(m1a, m1b), jnp.maximum(m2a, m2b)))


def _gate_block(x_ref, w_ref, b_ref, bias_ref, wout_ref, iout_ref):
    xb = x_ref[...]
    W = w_ref[...]
    s_lin = lax.dot_general(W, xb, (((1,), (1,)), ((), ())),
                            preferred_element_type=jnp.float32)
    s_lin = s_lin + b_ref[...]
    s2w = jax.nn.sigmoid(s_lin)
    score = s2w + bias_ref[...]
    bB = score.shape[1]
    neg = jnp.float32(-jnp.inf)

    gs_rows = []
    for g in range(NG):
        v = score[g * GSZ:(g + 1) * GSZ]
        m1, m2 = _top2_merge(v[0:4], jnp.full_like(v[0:4], neg),
                             v[4:8], jnp.full_like(v[0:4], neg))
        m1, m2 = _top2_merge(m1[0:2], m2[0:2], m1[2:4], m2[2:4])
        m1, m2 = _top2_merge(m1[0:1], m2[0:1], m1[1:2], m2[1:2])
        gs_rows.append(m1 + m2)
    gs = jnp.concatenate(gs_rows, axis=0)

    rowg = lax.broadcasted_iota(jnp.int32, (NG, bB), 0)
    rank = jnp.zeros((NG, bB), jnp.int32)
    for h in range(NG):
        gh = gs[h:h + 1]
        rank = rank + (gh > gs).astype(jnp.int32)
        rank = rank + ((gh == gs) & (h < rowg)).astype(jnp.int32)
    keep = (rank < KG).astype(jnp.float32)
    mask = jnp.concatenate(
        [jnp.broadcast_to(keep[g:g + 1], (GSZ, bB)) for g in range(NG)],
        axis=0)
    score_f = score * mask
    s2w_f = s2w * mask

    row = lax.broadcasted_iota(jnp.int32, (NE, bB), 0)
    cur = score_f
    wrows, irows = [], []
    for _ in range(TOPK):
        m = jnp.max(cur, axis=0, keepdims=True)
        lsel = jnp.min(jnp.where(cur == m, row, NE), axis=0, keepdims=True)
        hit = row == lsel
        wrows.append(jnp.max(jnp.where(hit, s2w_f, neg), axis=0, keepdims=True))
        irows.append(lsel)
        cur = jnp.where(hit, neg, cur)
    wout_ref[...] = jnp.concatenate(wrows, axis=0)
    iout_ref[...] = jnp.concatenate(irows, axis=0)


def kernel(x, W, b, bias):
    B = x.shape[0]
    bB = 1024
    b2 = b.reshape(NE, 1)
    bias2 = bias.reshape(NE, 1)
    wout, iout = pl.pallas_call(
        _gate_block,
        grid=(B // bB,),
        in_specs=[
            pl.BlockSpec((bB, DIN), lambda i: (i, 0)),
            pl.BlockSpec((NE, DIN), lambda i: (0, 0)),
            pl.BlockSpec((NE, 1), lambda i: (0, 0)),
            pl.BlockSpec((NE, 1), lambda i: (0, 0)),
        ],
        out_specs=[
            pl.BlockSpec((TOPK, bB), lambda i: (0, i)),
            pl.BlockSpec((TOPK, bB), lambda i: (0, i)),
        ],
        out_shape=[
            jax.ShapeDtypeStruct((TOPK, B), jnp.float32),
            jax.ShapeDtypeStruct((TOPK, B), jnp.int32),
        ],
    )(x, W, b2, bias2)
    return wout.T, iout.T


# speedup vs baseline: 13.2877x; 1.1238x over previous
import jax
import jax.numpy as jnp
from jax import lax
from jax.experimental import pallas as pl

TOPK = 8
NG = 8
GSZ = 8
KG = 4
NE = 64
DIN = 1024


def _top2_merge(m1a, m2a, m1b, m2b):
    return (jnp.maximum(m1a, m1b),
            jnp.maximum(jnp.minimum(m1a, m1b), jnp.maximum(m2a, m2b)))


def _gate_block(x_ref, w_ref, b_ref, bias_ref, wout_ref, iout_ref):
    xb = x_ref[...]
    W = w_ref[...]
    s_lin = lax.dot_general(W, xb, (((1,), (1,)), ((), ())),
                            preferred_element_type=jnp.float32)
    s_lin = s_lin + b_ref[...]
    s2w = jax.nn.sigmoid(s_lin)
    score = s2w + bias_ref[...]
    bB = score.shape[1]
    neg = jnp.float32(-jnp.inf)

    gs_rows = []
    for g in range(NG):
        v = score[g * GSZ:(g + 1) * GSZ]
        m1, m2 = _top2_merge(v[0:4], jnp.full_like(v[0:4], neg),
                             v[4:8], jnp.full_like(v[0:4], neg))
        m1, m2 = _top2_merge(m1[0:2], m2[0:2], m1[2:4], m2[2:4])
        m1, m2 = _top2_merge(m1[0:1], m2[0:1], m1[1:2], m2[1:2])
        gs_rows.append(m1 + m2)
    gs = jnp.concatenate(gs_rows, axis=0)

    rowg = lax.broadcasted_iota(jnp.int32, (NG, bB), 0)
    rank = jnp.zeros((NG, bB), jnp.int32)
    for h in range(NG):
        gh = gs[h:h + 1]
        rank = rank + (gh > gs).astype(jnp.int32)
        rank = rank + ((gh == gs) & (h < rowg)).astype(jnp.int32)
    keep = (rank < KG).astype(jnp.float32)
    mask = jnp.concatenate(
        [jnp.broadcast_to(keep[g:g + 1], (GSZ, bB)) for g in range(NG)],
        axis=0)
    score_f = score * mask
    s2w_f = s2w * mask

    row = lax.broadcasted_iota(jnp.int32, (NE, bB), 0)
    cur = score_f
    wrows, irows = [], []
    for _ in range(TOPK):
        m = jnp.max(cur, axis=0, keepdims=True)
        lsel = jnp.min(jnp.where(cur == m, row, NE), axis=0, keepdims=True)
        hit = row == lsel
        wrows.append(jnp.max(jnp.where(hit, s2w_f, neg), axis=0, keepdims=True))
        irows.append(lsel)
        cur = jnp.where(hit, neg, cur)
    wout_ref[...] = jnp.concatenate(wrows, axis=0)
    iout_ref[...] = jnp.concatenate(irows, axis=0)


def kernel(x, W, b, bias):
    B = x.shape[0]
    bB = 2048
    b2 = b.reshape(NE, 1)
    bias2 = bias.reshape(NE, 1)
    wout, iout = pl.pallas_call(
        _gate_block,
        grid=(B // bB,),
        in_specs=[
            pl.BlockSpec((bB, DIN), lambda i: (i, 0)),
            pl.BlockSpec((NE, DIN), lambda i: (0, 0)),
            pl.BlockSpec((NE, 1), lambda i: (0, 0)),
            pl.BlockSpec((NE, 1), lambda i: (0, 0)),
        ],
        out_specs=[
            pl.BlockSpec((TOPK, bB), lambda i: (0, i)),
            pl.BlockSpec((TOPK, bB), lambda i: (0, i)),
        ],
        out_shape=[
            jax.ShapeDtypeStruct((TOPK, B), jnp.float32),
            jax.ShapeDtypeStruct((TOPK, B), jnp.int32),
        ],
    )(x, W, b2, bias2)
    return wout.T, iout.T


# speedup vs baseline: 13.7818x; 1.0372x over previous
import jax
import jax.numpy as jnp
from jax import lax
from jax.experimental import pallas as pl

TOPK = 8
NG = 8
GSZ = 8
KG = 4
NE = 64
DIN = 1024


def _top2_merge(m1a, m2a, m1b, m2b):
    return (jnp.maximum(m1a, m1b),
            jnp.maximum(jnp.minimum(m1a, m1b), jnp.maximum(m2a, m2b)))


def _gate_block(x_ref, w_ref, b_ref, bias_ref, wout_ref, iout_ref):
    xb = x_ref[...]
    W = w_ref[...]
    s_lin = lax.dot_general(W, xb, (((1,), (1,)), ((), ())),
                            preferred_element_type=jnp.float32)
    s_lin = s_lin + b_ref[...]
    s2w = jax.nn.sigmoid(s_lin)
    score = s2w + bias_ref[...]
    bB = score.shape[1]
    neg = jnp.float32(-jnp.inf)

    gs_rows = []
    for g in range(NG):
        v = score[g * GSZ:(g + 1) * GSZ]
        m1, m2 = _top2_merge(v[0:4], jnp.full_like(v[0:4], neg),
                             v[4:8], jnp.full_like(v[0:4], neg))
        m1, m2 = _top2_merge(m1[0:2], m2[0:2], m1[2:4], m2[2:4])
        m1, m2 = _top2_merge(m1[0:1], m2[0:1], m1[1:2], m2[1:2])
        gs_rows.append(m1 + m2)
    gs = jnp.concatenate(gs_rows, axis=0)

    rowg = lax.broadcasted_iota(jnp.int32, (NG, bB), 0)
    rank = jnp.zeros((NG, bB), jnp.int32)
    for h in range(NG):
        gh = gs[h:h + 1]
        rank = rank + (gh > gs).astype(jnp.int32)
        rank = rank + ((gh == gs) & (h < rowg)).astype(jnp.int32)
    keep = (rank < KG).astype(jnp.float32)
    mask = jnp.concatenate(
        [jnp.broadcast_to(keep[g:g + 1], (GSZ, bB)) for g in range(NG)],
        axis=0)
    score_f = score * mask
    s2w_f = s2w * mask

    row = lax.broadcasted_iota(jnp.int32, (NE, bB), 0)
    cur = score_f
    wrows, irows = [], []
    for _ in range(TOPK):
        m = jnp.max(cur, axis=0, keepdims=True)
        lsel = jnp.min(jnp.where(cur == m, row, NE), axis=0, keepdims=True)
        hit = row == lsel
        wrows.append(jnp.max(jnp.where(hit, s2w_f, neg), axis=0, keepdims=True))
        irows.append(lsel)
        cur = jnp.where(hit, neg, cur)
    wout_ref[...] = jnp.concatenate(wrows, axis=0)
    iout_ref[...] = jnp.concatenate(irows, axis=0)


def kernel(x, W, b, bias):
    B = x.shape[0]
    bB = 4096
    b2 = b.reshape(NE, 1)
    bias2 = bias.reshape(NE, 1)
    wout, iout = pl.pallas_call(
        _gate_block,
        grid=(B // bB,),
        in_specs=[
            pl.BlockSpec((bB, DIN), lambda i: (i, 0)),
            pl.BlockSpec((NE, DIN), lambda i: (0, 0)),
            pl.BlockSpec((NE, 1), lambda i: (0, 0)),
            pl.BlockSpec((NE, 1), lambda i: (0, 0)),
        ],
        out_specs=[
            pl.BlockSpec((TOPK, bB), lambda i: (0, i)),
            pl.BlockSpec((TOPK, bB), lambda i: (0, i)),
        ],
        out_shape=[
            jax.ShapeDtypeStruct((TOPK, B), jnp.float32),
            jax.ShapeDtypeStruct((TOPK, B), jnp.int32),
        ],
    )(x, W, b2, bias2)
    return wout.T, iout.T
